# baseline (speedup 1.0000x reference)
import functools

import jax
import jax.numpy as jnp
from jax import lax
from jax.experimental import pallas as pl
from jax.experimental.pallas import tpu as pltpu
from jax.experimental.pallas import tpu_sc as plsc

B, L, D = 4096, 200, 64
V = 69
PAD_ID = 68
LANES = 16

NC, NS = 2, 16
NW = NC * NS
BW = B // NW
NBG = BW // LANES
DWAVE = 2


def _reg_take(vec, idx):
    dnums = lax.GatherDimensionNumbers(
        offset_dims=(), collapsed_slice_dims=(0,), start_index_map=(0,)
    )
    return lax.gather(
        vec, idx[:, None], dnums, slice_sizes=(1,),
        mode=lax.GatherScatterMode.PROMISE_IN_BOUNDS,
    )


def _mask_body(tok_ref, mask_ref):
    mask_ref[...] = tok_ref[...] == PAD_ID


_mask_kernel = pl.pallas_call(
    _mask_body,
    out_shape=jax.ShapeDtypeStruct((B, L), jnp.bool_),
)


@functools.cache
def _make_sc_embed():
    @functools.partial(
        pl.kernel,
        out_type=jax.ShapeDtypeStruct((L, D, B), jnp.float32),
        mesh=plsc.VectorSubcoreMesh(
            core_axis_name="c", subcore_axis_name="s",
            num_cores=NC, num_subcores=NS,
        ),
        scratch_types=[
            pltpu.VMEM((BW, L), jnp.int32),
            pltpu.VMEM((V * (D + 1),), jnp.float32),
            pltpu.VMEM((L * D,), jnp.float32),
            pltpu.VMEM((4, D, BW), jnp.float32),
            pltpu.SemaphoreType.DMA,
            pltpu.SemaphoreType.DMA,
            pltpu.SemaphoreType.DMA,
            pltpu.SemaphoreType.DMA,
        ],
        compiler_params=pltpu.CompilerParams(
            use_tc_tiling_on_sc=True, needs_layout_passes=False
        ),
    )
    def _sc_embed(tok_hbm, tab_hbm, pos_hbm, out_hbm, tok_v, tab_v, pos_v,
                  stage_v, sem0, sem1, sem2, sem3):
        wid = lax.axis_index("s") * NC + lax.axis_index("c")
        b0 = wid * BW
        sems = (sem0, sem1, sem2, sem3)

        pltpu.sync_copy(tok_hbm.at[pl.ds(b0, BW)], tok_v)
        pltpu.sync_copy(tab_hbm, tab_v)
        pltpu.sync_copy(pos_hbm, pos_v)

        base_iotas = [
            lax.iota(jnp.int32, LANES) + bg * LANES for bg in range(NBG)
        ]

        def wait_scatter(buf):
            pltpu.make_async_copy(
                stage_v.at[buf],
                out_hbm.at[0, :, pl.ds(0, BW)],
                sems[buf],
            ).wait()

        splat_idx = [
            jnp.full((LANES,), k, jnp.int32) for k in range(LANES)
        ]

        def compute_block(l, buf):
            lsplat = jnp.full((LANES,), l, jnp.int32)
            tokv65 = [
                plsc.load_gather(tok_v, [base_iotas[bg], lsplat]) * (D + 1)
                for bg in range(NBG)
            ]
            prow = [pos_v[pl.ds(l * D + k * LANES, LANES)] for k in range(4)]
            for d0 in range(0, D, DWAVE):
                ds = range(d0, d0 + DWAVE)
                pvs = {
                    d: _reg_take(prow[d // LANES], splat_idx[d % LANES])
                    for d in ds
                }
                es = {
                    (d, bg): plsc.load_gather(tab_v, [tokv65[bg] + d])
                    for d in ds
                    for bg in range(NBG)
                }
                for d in ds:
                    for bg in range(NBG):
                        stage_v[buf, d, bg * LANES:(bg + 1) * LANES] = (
                            es[d, bg] + pvs[d]
                        )

        def fire_scatter(l, buf):
            pltpu.async_copy(
                stage_v.at[buf],
                out_hbm.at[l, :, pl.ds(b0, BW)],
                sems[buf],
            )

        @pl.loop(0, L, step=4)
        def _quad(lo):
            for half in range(4):
                l = lo + half
                buf = half

                @pl.when(l >= 4)
                def _reuse_guard():
                    wait_scatter(buf)

                compute_block(l, buf)
                fire_scatter(l, buf)

        for buf in range(4):
            wait_scatter(buf)

    return _sc_embed


def kernel(token_indices, table, pos_enc):
    mask = _mask_kernel(token_indices)
    tab65 = jnp.pad(table, ((0, 0), (0, 1))).reshape(-1)
    out_t = _make_sc_embed()(
        token_indices, tab65, pos_enc[:L].reshape(-1)
    )
    emb = jnp.transpose(out_t, (2, 0, 1))
    return (emb, token_indices, mask)


# speedup vs baseline: 1.1455x; 1.1455x over previous
import functools

import jax
import jax.numpy as jnp
from jax import lax
from jax.experimental import pallas as pl
from jax.experimental.pallas import tpu as pltpu
from jax.experimental.pallas import tpu_sc as plsc

B, L, D = 4096, 200, 64
V = 69
PAD_ID = 68
LANES = 16

NC, NS = 2, 16
NW = NC * NS
BW = B // NW
NBG = BW // LANES
DWAVE = 2


def _reg_take(vec, idx):
    dnums = lax.GatherDimensionNumbers(
        offset_dims=(), collapsed_slice_dims=(0,), start_index_map=(0,)
    )
    return lax.gather(
        vec, idx[:, None], dnums, slice_sizes=(1,),
        mode=lax.GatherScatterMode.PROMISE_IN_BOUNDS,
    )


def _mask_body(tok_ref, mask_ref):
    mask_ref[...] = tok_ref[...] == PAD_ID


_mask_kernel = pl.pallas_call(
    _mask_body,
    out_shape=jax.ShapeDtypeStruct((B, L), jnp.bool_),
)


@functools.cache
def _make_sc_embed():
    @functools.partial(
        pl.kernel,
        out_type=jax.ShapeDtypeStruct((L, D, B), jnp.float32),
        mesh=plsc.VectorSubcoreMesh(
            core_axis_name="c", subcore_axis_name="s",
            num_cores=NC, num_subcores=NS,
        ),
        scratch_types=[
            pltpu.VMEM((BW, L), jnp.int32),
            pltpu.VMEM((V * (D + 1),), jnp.float32),
            pltpu.VMEM((L * D,), jnp.float32),
            pltpu.VMEM((2, D, BW), jnp.float32),
            pltpu.SemaphoreType.DMA,
            pltpu.SemaphoreType.DMA,
        ],
        compiler_params=pltpu.CompilerParams(
            use_tc_tiling_on_sc=True, needs_layout_passes=False
        ),
    )
    def _sc_embed(tok_hbm, tab_hbm, pos_hbm, out_hbm, tok_v, tab_v, pos_v,
                  stage_v, sem0, sem1):
        wid = lax.axis_index("s") * NC + lax.axis_index("c")
        b0 = wid * BW
        sems = (sem0, sem1)

        pltpu.sync_copy(tok_hbm.at[pl.ds(b0, BW)], tok_v)
        pltpu.sync_copy(tab_hbm, tab_v)
        pltpu.sync_copy(pos_hbm, pos_v)

        base_iotas = [
            lax.iota(jnp.int32, LANES) + bg * LANES for bg in range(NBG)
        ]

        def wait_scatter(buf):
            pltpu.make_async_copy(
                stage_v.at[buf],
                out_hbm.at[0, :, pl.ds(0, BW)],
                sems[buf],
            ).wait()

        splat_idx = [
            jnp.full((LANES,), k, jnp.int32) for k in range(LANES)
        ]

        def compute_block(l, buf):
            lsplat = jnp.full((LANES,), l, jnp.int32)
            tokv65 = [
                plsc.load_gather(tok_v, [base_iotas[bg], lsplat]) * (D + 1)
                for bg in range(NBG)
            ]
            prow = [pos_v[pl.ds(l * D + k * LANES, LANES)] for k in range(4)]
            for d0 in range(0, D, DWAVE):
                ds = range(d0, d0 + DWAVE)
                pvs = {
                    d: _reg_take(prow[d // LANES], splat_idx[d % LANES])
                    for d in ds
                }
                es = {
                    (d, bg): plsc.load_gather(tab_v, [tokv65[bg] + d])
                    for d in ds
                    for bg in range(NBG)
                }
                for d in ds:
                    for bg in range(NBG):
                        stage_v[buf, d, bg * LANES:(bg + 1) * LANES] = (
                            es[d, bg] + pvs[d]
                        )

        def fire_scatter(l, buf):
            pltpu.async_copy(
                stage_v.at[buf],
                out_hbm.at[l, :, pl.ds(b0, BW)],
                sems[buf],
            )

        for l0 in range(2):
            compute_block(l0, l0)
            fire_scatter(l0, l0)

        @pl.loop(2, L, step=2)
        def _pair(lo):
            for half in range(2):
                l = lo + half
                buf = half
                wait_scatter(buf)
                compute_block(l, buf)
                fire_scatter(l, buf)

        wait_scatter(0)
        wait_scatter(1)

    return _sc_embed


def kernel(token_indices, table, pos_enc):
    mask = _mask_kernel(token_indices)
    tab65 = jnp.pad(table, ((0, 0), (0, 1))).reshape(-1)
    out_t = _make_sc_embed()(
        token_indices, tab65, pos_enc[:L].reshape(-1)
    )
    emb = jnp.transpose(out_t, (2, 0, 1))
    return (emb, token_indices, mask)
